# Initial kernel scaffold; baseline (speedup 1.0000x reference)
import functools

import jax
import jax.numpy as jnp
from jax import lax
from jax.experimental import pallas as pl
from jax.experimental.pallas import tpu as pltpu
from jax.experimental.pallas import tpu_sc as plsc

N = 10000
E = 320000
D = 128

NC = 2
NS = 16
NW = NC * NS
EPW = E // NW
CHUNK = 80
NCHUNK = EPW // CHUNK
RPS = N // NS


def _sc_aggregate(feat, g, zeros):
    mesh = plsc.VectorSubcoreMesh(core_axis_name="c", subcore_axis_name="s")

    @functools.partial(
        pl.kernel,
        out_type=jax.ShapeDtypeStruct((NC * N, D), jnp.float32),
        mesh=mesh,
        scratch_types=[
            pltpu.VMEM((CHUNK,), jnp.int32),
            pltpu.VMEM((CHUNK,), jnp.int32),
            pltpu.VMEM((CHUNK, D), jnp.float32),
            pltpu.VMEM_SHARED((N, D), jnp.float32),
            pltpu.SemaphoreType.DMA,
        ],
    )
    def agg_kernel(feat_hbm, g_hbm, zeros_hbm, out_hbm, sidx, didx, rows, acc, sem):
        cid = lax.axis_index("c")
        sid = lax.axis_index("s")
        wid = sid * NC + cid

        pltpu.sync_copy(zeros_hbm, acc.at[pl.ds(sid * RPS, RPS)])
        plsc.subcore_barrier()

        base = wid * EPW

        @pl.loop(0, NCHUNK)
        def _(j):
            off = base + j * CHUNK
            pltpu.sync_copy(g_hbm.at[0, pl.ds(off, CHUNK)], sidx)
            pltpu.sync_copy(g_hbm.at[1, pl.ds(off, CHUNK)], didx)
            pltpu.async_copy(feat_hbm.at[sidx], rows, sem).wait()
            pltpu.sync_copy(rows, acc.at[didx], add=True)

        plsc.subcore_barrier()
        pltpu.sync_copy(
            acc.at[pl.ds(sid * RPS, RPS)],
            out_hbm.at[pl.ds(cid * N + sid * RPS, RPS)],
        )

    return agg_kernel(feat, g, zeros)


def _dense_layer(x, p0, p1, indeg, w_top, w_bot, b, gamma, beta, ln_relu):
    R = 2000

    def body(x_ref, p0_ref, p1_ref, d_ref, wt_ref, wb_ref, b_ref, g_ref,
             be_ref, o_ref):
        agg = (p0_ref[...] + p1_ref[...]) / d_ref[...]
        h = (
            jnp.dot(x_ref[...], wt_ref[...], preferred_element_type=jnp.float32)
            + jnp.dot(agg, wb_ref[...], preferred_element_type=jnp.float32)
            + b_ref[...]
        )
        if ln_relu:
            mu = jnp.mean(h, axis=-1, keepdims=True)
            var = jnp.mean((h - mu) ** 2, axis=-1, keepdims=True)
            h = (h - mu) * lax.rsqrt(var + 1e-5) * g_ref[...] + be_ref[...]
            h = jnp.maximum(h, 0.0)
        o_ref[...] = h

    row_spec = pl.BlockSpec((R, D), lambda i: (i, 0))
    full_spec = pl.BlockSpec((D, D), lambda i: (0, 0))
    vec_spec = pl.BlockSpec((1, D), lambda i: (0, 0))
    return pl.pallas_call(
        body,
        grid=(N // R,),
        in_specs=[
            row_spec, row_spec, row_spec,
            pl.BlockSpec((R, 1), lambda i: (i, 0)),
            full_spec, full_spec, vec_spec, vec_spec, vec_spec,
        ],
        out_specs=row_spec,
        out_shape=jax.ShapeDtypeStruct((N, D), jnp.float32),
    )(x, p0, p1, indeg, w_top, w_bot, b, gamma, beta)


def kernel(feat, g, in_deg, W1, b1, W2, b2, gamma, beta):
    zeros = jnp.zeros((RPS, D), jnp.float32)
    indeg = in_deg[:, None]
    b1r = b1[None, :]
    b2r = b2[None, :]
    gr = gamma[None, :]
    ber = beta[None, :]

    p = _sc_aggregate(feat, g, zeros)
    h1 = _dense_layer(feat, p[:N], p[N:], indeg, W1[:D], W1[D:], b1r, gr,
                      ber, True)
    p2 = _sc_aggregate(h1, g, zeros)
    return _dense_layer(h1, p2[:N], p2[N:], indeg, W2[:D], W2[D:], b2r, gr,
                        ber, False)


# speedup vs baseline: 5.0466x; 5.0466x over previous
import functools

import jax
import jax.numpy as jnp
from jax import lax
from jax.experimental import pallas as pl
from jax.experimental.pallas import tpu as pltpu
from jax.experimental.pallas import tpu_sc as plsc

N = 10000
E = 320000
D = 128

NC = 2
NS = 16
NW = NC * NS
EPW = E // NW
CHUNK = 80
NCHUNK = EPW // CHUNK
NPAD = 10240
RPS = NPAD // NS


def _sc_aggregate(feat, src, dst, zeros):
    mesh = plsc.VectorSubcoreMesh(core_axis_name="c", subcore_axis_name="s")

    @functools.partial(
        pl.kernel,
        out_type=jax.ShapeDtypeStruct((NC * NPAD, D), jnp.float32),
        mesh=mesh,
        scratch_types=[
            pltpu.VMEM((CHUNK,), jnp.int32),
            pltpu.VMEM((CHUNK,), jnp.int32),
            pltpu.VMEM((CHUNK, D), jnp.float32),
            pltpu.VMEM_SHARED((NPAD, D), jnp.float32),
            pltpu.SemaphoreType.DMA,
        ],
    )
    def agg_kernel(feat_hbm, src_hbm, dst_hbm, zeros_hbm, out_hbm, sidx, didx,
                   rows, acc, sem):
        cid = lax.axis_index("c")
        sid = lax.axis_index("s")
        wid = sid * NC + cid

        pltpu.sync_copy(zeros_hbm, acc.at[pl.ds(sid * RPS, RPS)])
        plsc.subcore_barrier()

        base = wid * EPW

        @pl.loop(0, NCHUNK)
        def _(j):
            off = base + j * CHUNK
            pltpu.sync_copy(src_hbm.at[pl.ds(off, CHUNK)], sidx)
            pltpu.sync_copy(dst_hbm.at[pl.ds(off, CHUNK)], didx)
            pltpu.async_copy(feat_hbm.at[sidx], rows, sem).wait()
            pltpu.sync_copy(rows, acc.at[didx], add=True)

        plsc.subcore_barrier()
        pltpu.sync_copy(
            acc.at[pl.ds(sid * RPS, RPS)],
            out_hbm.at[pl.ds(cid * NPAD + sid * RPS, RPS)],
        )

    return agg_kernel(feat, src, dst, zeros)


def _dense_layer(x, p0, p1, indeg, w_top, w_bot, b, gamma, beta, ln_relu):
    R = 2000

    def body(x_ref, p0_ref, p1_ref, d_ref, wt_ref, wb_ref, b_ref, g_ref,
             be_ref, o_ref):
        agg = (p0_ref[...] + p1_ref[...]) / d_ref[...]
        h = (
            jnp.dot(x_ref[...], wt_ref[...], preferred_element_type=jnp.float32)
            + jnp.dot(agg, wb_ref[...], preferred_element_type=jnp.float32)
            + b_ref[...]
        )
        if ln_relu:
            mu = jnp.mean(h, axis=-1, keepdims=True)
            var = jnp.mean((h - mu) ** 2, axis=-1, keepdims=True)
            h = (h - mu) * lax.rsqrt(var + 1e-5) * g_ref[...] + be_ref[...]
            h = jnp.maximum(h, 0.0)
        o_ref[...] = h

    row_spec = pl.BlockSpec((R, D), lambda i: (i, 0))
    full_spec = pl.BlockSpec((D, D), lambda i: (0, 0))
    vec_spec = pl.BlockSpec((1, D), lambda i: (0, 0))
    return pl.pallas_call(
        body,
        grid=(N // R,),
        in_specs=[
            row_spec, row_spec, row_spec,
            pl.BlockSpec((R, 1), lambda i: (i, 0)),
            full_spec, full_spec, vec_spec, vec_spec, vec_spec,
        ],
        out_specs=row_spec,
        out_shape=jax.ShapeDtypeStruct((N, D), jnp.float32),
    )(x, p0, p1, indeg, w_top, w_bot, b, gamma, beta)


def kernel(feat, g, in_deg, W1, b1, W2, b2, gamma, beta):
    zeros = jnp.zeros((RPS, D), jnp.float32)
    indeg = in_deg[:, None]
    b1r = b1[None, :]
    b2r = b2[None, :]
    gr = gamma[None, :]
    ber = beta[None, :]

    src = g[0]
    dst = g[1]
    p = _sc_aggregate(feat, src, dst, zeros)
    h1 = _dense_layer(feat, p[:N], p[NPAD:NPAD + N], indeg, W1[:D], W1[D:],
                      b1r, gr, ber, True)
    p2 = _sc_aggregate(h1, src, dst, zeros)
    return _dense_layer(h1, p2[:N], p2[NPAD:NPAD + N], indeg, W2[:D], W2[D:],
                        b2r, gr, ber, False)
